# Initial kernel scaffold; baseline (speedup 1.0000x reference)
import jax
import jax.numpy as jnp
from jax.experimental import pallas as pl


def kernel(x, y):
    raise NotImplementedError("write your pallas kernel here")


# speedup vs baseline: 81.3661x; 81.3661x over previous
import functools

import jax
import jax.numpy as jnp
from jax import lax
from jax.experimental import pallas as pl
from jax.experimental.pallas import tpu as pltpu
from jax.experimental.pallas import tpu_sc as plsc

_K = 5
_L = 16
_NC = 2
_NS = 16
_NW = _NC * _NS
_B = 1024
_N = 100000
_RW = _B // _NW
_CHUNKS = _N // _L


def _body(x_hbm, y_hbm, out_hbm, row_v, y_v, out_v, sem):
    wid = lax.axis_index("s") * _NC + lax.axis_index("c")
    base = wid * _RW

    pltpu.sync_copy(y_hbm.at[pl.ds(base, _RW)], y_v)

    lanes = lax.iota(jnp.int32, _L)
    lane0 = lanes == 0
    neg_inf = jnp.float32(-jnp.inf)

    def row_loop(j, loss_acc):
        row = base + j
        pltpu.async_copy(x_hbm.at[row], row_v, sem).wait()

        y_vec = y_v[pl.ds((j // _L) * _L, _L)]
        y_i = jnp.sum(jnp.where(lanes == (j % _L), y_vec, 0))
        c_y = y_i // _L
        l_y = y_i % _L
        vy = row_v[pl.ds(c_y * _L, _L)]
        eq = lanes == l_y
        s_y = jnp.sum(jnp.where(eq, vy, 0.0))
        row_v[pl.ds(c_y * _L, _L)] = jnp.where(eq, 0.0, vy)

        def chunk_body(c, carry):
            t1, t2, t3, t4, t5 = carry
            v = row_v[pl.ds(c * _L, _L)]
            m1 = jnp.maximum(t1, v)
            c1 = jnp.minimum(t1, v)
            m2 = jnp.maximum(t2, c1)
            c2 = jnp.minimum(t2, c1)
            m3 = jnp.maximum(t3, c2)
            c3 = jnp.minimum(t3, c2)
            m4 = jnp.maximum(t4, c3)
            c4 = jnp.minimum(t4, c3)
            m5 = jnp.maximum(t5, c4)
            return (m1, m2, m3, m4, m5)

        init = tuple(jnp.full((_L,), neg_inf) for _ in range(_K))
        t1, t2, t3, t4, t5 = lax.fori_loop(0, _CHUNKS, chunk_body, init,
                                           unroll=8)

        for _ in range(_K - 1):
            m = jnp.max(t1)
            ffs = plsc.all_reduce_ffs(t1 == m)
            sel = lanes == ffs
            t1 = jnp.where(sel, t2, t1)
            t2 = jnp.where(sel, t3, t2)
            t3 = jnp.where(sel, t4, t3)
            t4 = jnp.where(sel, t5, t4)
            t5 = jnp.where(sel, neg_inf, t5)
        s_topk = jnp.max(t1)

        hinge = jnp.maximum(1.0 + s_topk - s_y, 0.0)
        return loss_acc + jnp.where(lane0, hinge, 0.0)

    loss_acc = lax.fori_loop(0, _RW, row_loop, jnp.zeros((_L,), jnp.float32))

    out_v[...] = loss_acc
    pltpu.sync_copy(out_v, out_hbm.at[wid])


@jax.jit
def kernel(x, y):
    mesh = plsc.VectorSubcoreMesh(core_axis_name="c", subcore_axis_name="s")
    partials = pl.kernel(
        _body,
        out_type=jax.ShapeDtypeStruct((_NW, _L), jnp.float32),
        mesh=mesh,
        compiler_params=pltpu.CompilerParams(needs_layout_passes=False),
        scratch_types=[
            pltpu.VMEM((_N,), jnp.float32),
            pltpu.VMEM((_RW,), jnp.int32),
            pltpu.VMEM((_L,), jnp.float32),
            pltpu.SemaphoreType.DMA,
        ],
    )(x, y)
    return jnp.sum(partials[:, 0]) / jnp.float32(_B)
